# baseline (speedup 1.0000x reference)
import functools

import jax
import jax.numpy as jnp
from jax import lax
from jax.experimental import pallas as pl
from jax.experimental.pallas import tpu as pltpu
from jax.experimental.pallas import tpu_sc as plsc

NC = 2
NS = 16
NW = NC * NS
CH = 128
_CHUNKS_C0 = 40


@functools.partial(jax.jit,
                   static_argnames=("n_chunks", "n_pad", "ch", "tiled",
                                    "chunks_c0"))
def _sc_scatter(y, src3, dst3, zeros, n_chunks, n_pad, ch, tiled,
                chunks_c0=None):
    n, h = y.shape
    rows_per_tile = n_pad // NS
    cmax = src3.shape[1]
    mesh = plsc.VectorSubcoreMesh(core_axis_name="c", subcore_axis_name="s")

    @functools.partial(
        pl.kernel,
        out_type=jax.ShapeDtypeStruct((NC, n_pad, h), jnp.float32),
        mesh=mesh,
        scratch_types=[
            pltpu.VMEM((cmax, ch), jnp.int32),
            pltpu.VMEM((cmax, ch), jnp.int32),
            pltpu.VMEM((ch, h), jnp.float32),
            pltpu.VMEM_SHARED((n_pad, h), jnp.float32),
            pltpu.SemaphoreType.DMA,
        ],
        compiler_params=pltpu.CompilerParams(use_tc_tiling_on_sc=tiled),
    )
    def sc_kernel(y_hbm, src_hbm, dst_hbm, zeros_hbm, out_hbm,
                  src_v, dst_v, rows_v, acc_sh, gsem):
        cid = lax.axis_index("c")
        sid = lax.axis_index("s")
        wid = sid * NC + cid
        row0 = sid * rows_per_tile
        pltpu.sync_copy(zeros_hbm, acc_sh.at[pl.ds(row0, rows_per_tile)])
        pltpu.sync_copy(src_hbm.at[wid], src_v)
        pltpu.sync_copy(dst_hbm.at[wid], dst_v)
        plsc.subcore_barrier()

        def body(j, carry):
            pltpu.async_copy(y_hbm.at[src_v.at[j]], rows_v, gsem).wait()
            pltpu.sync_copy(rows_v, acc_sh.at[dst_v.at[j]], add=True)
            return carry

        if chunks_c0 is None:
            lax.fori_loop(0, n_chunks, body, 0)
        else:
            bound = jnp.where(cid == 0, chunks_c0,
                              2 * n_chunks - chunks_c0)
            lax.fori_loop(0, bound, body, 0)
        plsc.subcore_barrier()
        pltpu.sync_copy(acc_sh.at[pl.ds(row0, rows_per_tile)],
                        out_hbm.at[cid, pl.ds(row0, rows_per_tile)])

    return sc_kernel(y, src3, dst3, zeros)


@functools.partial(jax.jit, static_argnames=("n_chunks", "n_pad"))
def _sc_half(y, src3, dst3, zeros, n_chunks, n_pad):
    n, h = y.shape
    rows_per_tile = n_pad // NS
    cmax = src3.shape[1]
    ch = src3.shape[2]
    mesh = plsc.VectorSubcoreMesh(core_axis_name="c", subcore_axis_name="s")

    @functools.partial(
        pl.kernel,
        out_type=jax.ShapeDtypeStruct((NC, n_pad, h), jnp.float32),
        mesh=mesh,
        scratch_types=[
            pltpu.VMEM((cmax // 2, ch), jnp.int32),
            pltpu.VMEM((cmax // 2, ch), jnp.int32),
            pltpu.VMEM((ch, h), jnp.float32),
            pltpu.VMEM_SHARED((n_pad, h), jnp.float32),
            pltpu.VMEM_SHARED((n_pad, h), jnp.float32),
            pltpu.SemaphoreType.DMA,
        ],
        compiler_params=pltpu.CompilerParams(use_tc_tiling_on_sc=False),
    )
    def sc_kernel(y_hbm, src_hbm, dst_hbm, zeros_hbm, out_hbm,
                  src_v, dst_v, rows_v, y_sh, acc_sh, gsem):
        cid = lax.axis_index("c")
        sid = lax.axis_index("s")
        wid = sid * NC + cid
        row0 = sid * rows_per_tile
        pltpu.sync_copy(y_hbm.at[pl.ds(row0, rows_per_tile)],
                        y_sh.at[pl.ds(row0, rows_per_tile)])
        pltpu.sync_copy(zeros_hbm, acc_sh.at[pl.ds(row0, rows_per_tile)])
        plsc.subcore_barrier()

        def body(j, carry):
            pltpu.async_copy(y_sh.at[src_v.at[j]], rows_v, gsem).wait()
            pltpu.sync_copy(rows_v, acc_sh.at[dst_v.at[j]], add=True)
            return carry

        n_half = n_chunks // 2
        for base in (0, n_half):
            pltpu.sync_copy(src_hbm.at[wid, pl.ds(base, n_half)], src_v)
            pltpu.sync_copy(dst_hbm.at[wid, pl.ds(base, n_half)], dst_v)
            lax.fori_loop(0, n_half, body, 0)
        plsc.subcore_barrier()
        pltpu.sync_copy(acc_sh.at[pl.ds(row0, rows_per_tile)],
                        out_hbm.at[cid, pl.ds(row0, rows_per_tile)])

    return sc_kernel(y, src3, dst3, zeros)


@functools.partial(jax.jit, static_argnames=("n_chunks", "n_pad"))
def _sc_layer(ylo, yhi, src3, dst3, zeros, n_chunks, n_pad):
    n, h = ylo.shape
    rows_per_tile = n_pad // NS
    cmax = src3.shape[1]
    ch = src3.shape[2]
    mesh = plsc.VectorSubcoreMesh(core_axis_name="c", subcore_axis_name="s")

    out_t = jax.ShapeDtypeStruct((NC, n_pad, h), jnp.float32)

    @functools.partial(
        pl.kernel,
        out_type=(out_t, out_t),
        mesh=mesh,
        scratch_types=[
            pltpu.VMEM((cmax // 2, ch), jnp.int32),
            pltpu.VMEM((cmax // 2, ch), jnp.int32),
            pltpu.VMEM((ch, h), jnp.float32),
            pltpu.VMEM_SHARED((n_pad, h), jnp.float32),
            pltpu.VMEM_SHARED((n_pad, h), jnp.float32),
            pltpu.SemaphoreType.DMA,
        ],
        compiler_params=pltpu.CompilerParams(use_tc_tiling_on_sc=False),
    )
    def sc_kernel(ylo_hbm, yhi_hbm, src_hbm, dst_hbm, zeros_hbm,
                  alo_hbm, ahi_hbm,
                  src_v, dst_v, rows_v, y_sh, acc_sh, gsem):
        cid = lax.axis_index("c")
        sid = lax.axis_index("s")
        wid = sid * NC + cid
        row0 = sid * rows_per_tile
        rows = pl.ds(row0, rows_per_tile)
        n_half = n_chunks // 2

        def body(j, carry):
            pltpu.async_copy(y_sh.at[src_v.at[j]], rows_v, gsem).wait()
            pltpu.sync_copy(rows_v, acc_sh.at[dst_v.at[j]], add=True)
            return carry

        for y_hbm, out_hbm in ((ylo_hbm, alo_hbm), (yhi_hbm, ahi_hbm)):
            pltpu.sync_copy(y_hbm.at[rows], y_sh.at[rows])
            pltpu.sync_copy(zeros_hbm, acc_sh.at[rows])
            plsc.subcore_barrier()
            for base in (0, n_half):
                pltpu.sync_copy(src_hbm.at[wid, pl.ds(base, n_half)], src_v)
                pltpu.sync_copy(dst_hbm.at[wid, pl.ds(base, n_half)], dst_v)
                lax.fori_loop(0, n_half, body, 0)
            plsc.subcore_barrier()
            pltpu.sync_copy(acc_sh.at[rows], out_hbm.at[cid, rows])

    return sc_kernel(ylo, yhi, src3, dst3, zeros)


_BN = 632


def _tc_matmul(x, w):
    n, d = x.shape
    h = w.shape[1]

    def body(x_ref, w_ref, o_ref):
        o_ref[...] = jnp.dot(x_ref[...], w_ref[...],
                             preferred_element_type=jnp.float32)

    return pl.pallas_call(
        body,
        grid=(n // _BN,),
        in_specs=[pl.BlockSpec((_BN, d), lambda i: (i, 0)),
                  pl.BlockSpec((d, h), lambda i: (0, 0))],
        out_specs=pl.BlockSpec((_BN, h), lambda i: (i, 0)),
        out_shape=jax.ShapeDtypeStruct((n, h), jnp.float32),
    )(x, w)


def _tc_fuse(y, a0, a1, b, w):
    n, d = y.shape
    h = w.shape[1]

    def body(y_ref, a0_ref, a1_ref, b_ref, w_ref, o_ref):
        z = jnp.maximum(y_ref[...] + a0_ref[...] + a1_ref[...] + b_ref[...], 0.0)
        o_ref[...] = jnp.dot(z, w_ref[...], preferred_element_type=jnp.float32)

    return pl.pallas_call(
        body,
        grid=(n // _BN,),
        in_specs=[pl.BlockSpec((_BN, d), lambda i: (i, 0)),
                  pl.BlockSpec((_BN, d), lambda i: (i, 0)),
                  pl.BlockSpec((_BN, d), lambda i: (i, 0)),
                  pl.BlockSpec((1, d), lambda i: (0, 0)),
                  pl.BlockSpec((d, h), lambda i: (0, 0))],
        out_specs=pl.BlockSpec((_BN, h), lambda i: (i, 0)),
        out_shape=jax.ShapeDtypeStruct((n, h), jnp.float32),
    )(y, a0, a1, b, w)


def _tc_matmul2(x, w):
    n, d = x.shape
    h = w.shape[1]
    hh = h // 2

    def body(x_ref, w_ref, o1_ref, o2_ref):
        t = jnp.dot(x_ref[...], w_ref[...], preferred_element_type=jnp.float32)
        o1_ref[...] = t[:, :hh]
        o2_ref[...] = t[:, hh:]

    return pl.pallas_call(
        body,
        grid=(n // _BN,),
        in_specs=[pl.BlockSpec((_BN, d), lambda i: (i, 0)),
                  pl.BlockSpec((d, h), lambda i: (0, 0))],
        out_specs=[pl.BlockSpec((_BN, hh), lambda i: (i, 0)),
                   pl.BlockSpec((_BN, hh), lambda i: (i, 0))],
        out_shape=[jax.ShapeDtypeStruct((n, hh), jnp.float32),
                   jax.ShapeDtypeStruct((n, hh), jnp.float32)],
    )(x, w)


def _tc_fuse3(ylo, yhi, alo0, alo1, ahi0, ahi1, b, w):
    n, hh = ylo.shape
    h = w.shape[1]

    def body(ylo_ref, yhi_ref, alo0_ref, alo1_ref, ahi0_ref, ahi1_ref,
             b_ref, w_ref, o1_ref, o2_ref):
        zlo = ylo_ref[...] + alo0_ref[...] + alo1_ref[...] + b_ref[:, :hh]
        zhi = yhi_ref[...] + ahi0_ref[...] + ahi1_ref[...] + b_ref[:, hh:]
        z = jnp.maximum(jnp.concatenate([zlo, zhi], axis=1), 0.0)
        t = jnp.dot(z, w_ref[...], preferred_element_type=jnp.float32)
        o1_ref[...] = t[:, :hh]
        o2_ref[...] = t[:, hh:]

    hspec = pl.BlockSpec((_BN, hh), lambda i: (i, 0))
    return pl.pallas_call(
        body,
        grid=(n // _BN,),
        in_specs=[hspec, hspec, hspec, hspec, hspec, hspec,
                  pl.BlockSpec((1, 2 * hh), lambda i: (0, 0)),
                  pl.BlockSpec((2 * hh, h), lambda i: (0, 0))],
        out_specs=[hspec, hspec],
        out_shape=[jax.ShapeDtypeStruct((n, hh), jnp.float32),
                   jax.ShapeDtypeStruct((n, hh), jnp.float32)],
    )(ylo, yhi, alo0, alo1, ahi0, ahi1, b, w)


def _tc_fuse4(ylo, yhi, alo0, alo1, ahi0, ahi1, b, w3, wl):
    n, hh = ylo.shape
    l = wl.shape[1]

    def body(ylo_ref, yhi_ref, alo0_ref, alo1_ref, ahi0_ref, ahi1_ref,
             b_ref, w3_ref, wl_ref, o_ref):
        zlo = ylo_ref[...] + alo0_ref[...] + alo1_ref[...] + b_ref[:, :hh]
        zhi = yhi_ref[...] + ahi0_ref[...] + ahi1_ref[...] + b_ref[:, hh:]
        z = jnp.maximum(jnp.concatenate([zlo, zhi], axis=1), 0.0)
        w = jnp.dot(w3_ref[...], wl_ref[...], preferred_element_type=jnp.float32)
        o_ref[...] = jnp.dot(z, w, preferred_element_type=jnp.float32)

    hspec = pl.BlockSpec((_BN, hh), lambda i: (i, 0))
    return pl.pallas_call(
        body,
        grid=(n // _BN,),
        in_specs=[hspec, hspec, hspec, hspec, hspec, hspec,
                  pl.BlockSpec((1, 2 * hh), lambda i: (0, 0)),
                  pl.BlockSpec((2 * hh, l), lambda i: (0, 0)),
                  pl.BlockSpec((l, l), lambda i: (0, 0))],
        out_specs=pl.BlockSpec((_BN, l), lambda i: (i, 0)),
        out_shape=jax.ShapeDtypeStruct((n, l), jnp.float32),
    )(ylo, yhi, alo0, alo1, ahi0, ahi1, b, w3, wl)


def _tc_fuse2(y, a0, a1, b, w3, wl):
    n, d = y.shape
    l = wl.shape[1]

    def body(y_ref, a0_ref, a1_ref, b_ref, w3_ref, wl_ref, o_ref):
        z = jnp.maximum(y_ref[...] + a0_ref[...] + a1_ref[...] + b_ref[...], 0.0)
        w = jnp.dot(w3_ref[...], wl_ref[...], preferred_element_type=jnp.float32)
        o_ref[...] = jnp.dot(z, w, preferred_element_type=jnp.float32)

    return pl.pallas_call(
        body,
        grid=(n // _BN,),
        in_specs=[pl.BlockSpec((_BN, d), lambda i: (i, 0)),
                  pl.BlockSpec((_BN, d), lambda i: (i, 0)),
                  pl.BlockSpec((_BN, d), lambda i: (i, 0)),
                  pl.BlockSpec((1, d), lambda i: (0, 0)),
                  pl.BlockSpec((d, l), lambda i: (0, 0)),
                  pl.BlockSpec((l, l), lambda i: (0, 0))],
        out_specs=pl.BlockSpec((_BN, l), lambda i: (i, 0)),
        out_shape=jax.ShapeDtypeStruct((n, l), jnp.float32),
    )(y, a0, a1, b, w3, wl)


def _tc_final(y, a0, a1, b3, wl, bl):
    n, l = y.shape

    def body(y_ref, a0_ref, a1_ref, b3_ref, wl_ref, bl_ref, o_ref):
        bp = jnp.dot(b3_ref[...], wl_ref[...],
                     preferred_element_type=jnp.float32) + bl_ref[...]
        o_ref[...] = y_ref[...] + a0_ref[...] + a1_ref[...] + bp

    return pl.pallas_call(
        body,
        grid=(n // _BN,),
        in_specs=[pl.BlockSpec((_BN, l), lambda i: (i, 0)),
                  pl.BlockSpec((_BN, l), lambda i: (i, 0)),
                  pl.BlockSpec((_BN, l), lambda i: (i, 0)),
                  pl.BlockSpec((1, l), lambda i: (0, 0)),
                  pl.BlockSpec((l, l), lambda i: (0, 0)),
                  pl.BlockSpec((1, l), lambda i: (0, 0))],
        out_specs=pl.BlockSpec((_BN, l), lambda i: (i, 0)),
        out_shape=jax.ShapeDtypeStruct((n, l), jnp.float32),
    )(y, a0, a1, b3, wl, bl)


def kernel(x, edge_index, batch, W1, b1, W2, b2, W3, b3, Wl, bl):
    n, d = x.shape
    e = edge_index.shape[1]
    src = edge_index[0]
    dst = edge_index[1]

    per_tile = -(-e // NW)
    per_tile = -(-per_tile // 1024) * 1024
    n_chunks = per_tile // CH
    e_pad = NW * per_tile
    n_pad = -(-n // (NS * 8)) * (NS * 8)
    src_p = jnp.concatenate([src, jnp.zeros((e_pad - e,), jnp.int32)])
    dst_p = jnp.concatenate([dst, jnp.full((e_pad - e,), n, jnp.int32)])
    src3w = src_p.reshape(NW, per_tile // 512, 512)
    dst3w = dst_p.reshape(NW, per_tile // 512, 512)

    x_p = jnp.concatenate([x, jnp.zeros((n_pad - n, d), jnp.float32)])

    rows_per_tile = n_pad // NS
    zeros_hh = jnp.zeros((rows_per_tile, W1.shape[1] // 2), jnp.float32)
    zeros_l = jnp.zeros((rows_per_tile, Wl.shape[0]), jnp.float32)

    b1r = b1.reshape(1, -1)
    b2r = b2.reshape(1, -1)
    b3r = b3.reshape(1, -1)
    blr = bl.reshape(1, -1)

    nck = per_tile // 512

    y1lo, y1hi = _tc_matmul2(x_p, W1)
    a1lo, a1hi = _sc_layer(y1lo, y1hi, src3w, dst3w, zeros_hh,
                           n_chunks=nck, n_pad=n_pad)
    y2lo, y2hi = _tc_fuse3(y1lo, y1hi, a1lo[0], a1lo[1], a1hi[0], a1hi[1],
                           b1r, W2)
    a2lo, a2hi = _sc_layer(y2lo, y2hi, src3w, dst3w, zeros_hh,
                           n_chunks=nck, n_pad=n_pad)
    y3 = _tc_fuse4(y2lo, y2hi, a2lo[0], a2lo[1], a2hi[0], a2hi[1],
                   b2r, W3, Wl)
    a3 = _sc_half(y3, src3w, dst3w, zeros_l, n_chunks=nck, n_pad=n_pad)
    return _tc_final(y3, a3[0], a3[1], b3r, Wl, blr)[:n]


# speedup vs baseline: 1.0022x; 1.0022x over previous
import functools

import jax
import jax.numpy as jnp
from jax import lax
from jax.experimental import pallas as pl
from jax.experimental.pallas import tpu as pltpu
from jax.experimental.pallas import tpu_sc as plsc

NC = 2
NS = 16
NW = NC * NS
CH = 512


@functools.partial(jax.jit, static_argnames=("n_chunks", "n_pad"))
def _sc_half(y, src3, dst3, zeros, n_chunks, n_pad):
    n, h = y.shape
    rows_per_tile = n_pad // NS
    cmax = src3.shape[1]
    ch = src3.shape[2]
    mesh = plsc.VectorSubcoreMesh(core_axis_name="c", subcore_axis_name="s")

    @functools.partial(
        pl.kernel,
        out_type=jax.ShapeDtypeStruct((NC, n_pad, h), jnp.float32),
        mesh=mesh,
        scratch_types=[
            pltpu.VMEM((cmax // 2, ch), jnp.int32),
            pltpu.VMEM((cmax // 2, ch), jnp.int32),
            pltpu.VMEM((ch, h), jnp.float32),
            pltpu.VMEM_SHARED((n_pad, h), jnp.float32),
            pltpu.VMEM_SHARED((n_pad, h), jnp.float32),
            pltpu.SemaphoreType.DMA,
        ],
        compiler_params=pltpu.CompilerParams(use_tc_tiling_on_sc=False),
    )
    def sc_kernel(y_hbm, src_hbm, dst_hbm, zeros_hbm, out_hbm,
                  src_v, dst_v, rows_v, y_sh, acc_sh, gsem):
        cid = lax.axis_index("c")
        sid = lax.axis_index("s")
        wid = sid * NC + cid
        rows = pl.ds(sid * rows_per_tile, rows_per_tile)
        pltpu.sync_copy(y_hbm.at[rows], y_sh.at[rows])
        pltpu.sync_copy(zeros_hbm, acc_sh.at[rows])
        plsc.subcore_barrier()

        def body(j, carry):
            pltpu.async_copy(y_sh.at[src_v.at[j]], rows_v, gsem).wait()
            pltpu.sync_copy(rows_v, acc_sh.at[dst_v.at[j]], add=True)
            return carry

        n_half = n_chunks // 2
        for base in (0, n_half):
            pltpu.sync_copy(src_hbm.at[wid, pl.ds(base, n_half)], src_v)
            pltpu.sync_copy(dst_hbm.at[wid, pl.ds(base, n_half)], dst_v)
            lax.fori_loop(0, n_half, body, 0)
        plsc.subcore_barrier()
        pltpu.sync_copy(acc_sh.at[rows], out_hbm.at[cid, rows])

    return sc_kernel(y, src3, dst3, zeros)


@functools.partial(jax.jit, static_argnames=("n_chunks", "n_pad"))
def _sc_layer(ylo, yhi, src3, dst3, zeros, n_chunks, n_pad):
    n, h = ylo.shape
    rows_per_tile = n_pad // NS
    cmax = src3.shape[1]
    ch = src3.shape[2]
    mesh = plsc.VectorSubcoreMesh(core_axis_name="c", subcore_axis_name="s")

    out_t = jax.ShapeDtypeStruct((NC, n_pad, h), jnp.float32)

    @functools.partial(
        pl.kernel,
        out_type=(out_t, out_t),
        mesh=mesh,
        scratch_types=[
            pltpu.VMEM((cmax // 2, ch), jnp.int32),
            pltpu.VMEM((cmax // 2, ch), jnp.int32),
            pltpu.VMEM((ch, h), jnp.float32),
            pltpu.VMEM_SHARED((n_pad, h), jnp.float32),
            pltpu.VMEM_SHARED((n_pad, h), jnp.float32),
            pltpu.SemaphoreType.DMA,
        ],
        compiler_params=pltpu.CompilerParams(use_tc_tiling_on_sc=False),
    )
    def sc_kernel(ylo_hbm, yhi_hbm, src_hbm, dst_hbm, zeros_hbm,
                  alo_hbm, ahi_hbm,
                  src_v, dst_v, rows_v, y_sh, acc_sh, gsem):
        cid = lax.axis_index("c")
        sid = lax.axis_index("s")
        wid = sid * NC + cid
        rows = pl.ds(sid * rows_per_tile, rows_per_tile)
        n_half = n_chunks // 2

        def body(j, carry):
            pltpu.async_copy(y_sh.at[src_v.at[j]], rows_v, gsem).wait()
            pltpu.sync_copy(rows_v, acc_sh.at[dst_v.at[j]], add=True)
            return carry

        for y_hbm, out_hbm in ((ylo_hbm, alo_hbm), (yhi_hbm, ahi_hbm)):
            pltpu.sync_copy(y_hbm.at[rows], y_sh.at[rows])
            pltpu.sync_copy(zeros_hbm, acc_sh.at[rows])
            plsc.subcore_barrier()
            for base in (0, n_half):
                pltpu.sync_copy(src_hbm.at[wid, pl.ds(base, n_half)], src_v)
                pltpu.sync_copy(dst_hbm.at[wid, pl.ds(base, n_half)], dst_v)
                lax.fori_loop(0, n_half, body, 0)
            plsc.subcore_barrier()
            pltpu.sync_copy(acc_sh.at[rows], out_hbm.at[cid, rows])

    return sc_kernel(ylo, yhi, src3, dst3, zeros)


_BN = 632


def _tc_matmul2(x, w):
    n, d = x.shape
    h = w.shape[1]
    hh = h // 2

    def body(x_ref, w_ref, o1_ref, o2_ref):
        t = jnp.dot(x_ref[...], w_ref[...], preferred_element_type=jnp.float32)
        o1_ref[...] = t[:, :hh]
        o2_ref[...] = t[:, hh:]

    return pl.pallas_call(
        body,
        grid=(n // _BN,),
        in_specs=[pl.BlockSpec((_BN, d), lambda i: (i, 0)),
                  pl.BlockSpec((d, h), lambda i: (0, 0))],
        out_specs=[pl.BlockSpec((_BN, hh), lambda i: (i, 0)),
                   pl.BlockSpec((_BN, hh), lambda i: (i, 0))],
        out_shape=[jax.ShapeDtypeStruct((n, hh), jnp.float32),
                   jax.ShapeDtypeStruct((n, hh), jnp.float32)],
    )(x, w)


def _tc_fuse3(ylo, yhi, alo0, alo1, ahi0, ahi1, b, w):
    n, hh = ylo.shape
    h = w.shape[1]

    def body(ylo_ref, yhi_ref, alo0_ref, alo1_ref, ahi0_ref, ahi1_ref,
             b_ref, w_ref, o1_ref, o2_ref):
        zlo = ylo_ref[...] + alo0_ref[...] + alo1_ref[...] + b_ref[:, :hh]
        zhi = yhi_ref[...] + ahi0_ref[...] + ahi1_ref[...] + b_ref[:, hh:]
        z = jnp.maximum(jnp.concatenate([zlo, zhi], axis=1), 0.0)
        t = jnp.dot(z, w_ref[...], preferred_element_type=jnp.float32)
        o1_ref[...] = t[:, :hh]
        o2_ref[...] = t[:, hh:]

    hspec = pl.BlockSpec((_BN, hh), lambda i: (i, 0))
    return pl.pallas_call(
        body,
        grid=(n // _BN,),
        in_specs=[hspec, hspec, hspec, hspec, hspec, hspec,
                  pl.BlockSpec((1, 2 * hh), lambda i: (0, 0)),
                  pl.BlockSpec((2 * hh, h), lambda i: (0, 0))],
        out_specs=[hspec, hspec],
        out_shape=[jax.ShapeDtypeStruct((n, hh), jnp.float32),
                   jax.ShapeDtypeStruct((n, hh), jnp.float32)],
    )(ylo, yhi, alo0, alo1, ahi0, ahi1, b, w)


def _tc_fuse4(ylo, yhi, alo0, alo1, ahi0, ahi1, b, w3, wl):
    n, hh = ylo.shape
    l = wl.shape[1]

    def body(ylo_ref, yhi_ref, alo0_ref, alo1_ref, ahi0_ref, ahi1_ref,
             b_ref, w3_ref, wl_ref, o_ref):
        zlo = ylo_ref[...] + alo0_ref[...] + alo1_ref[...] + b_ref[:, :hh]
        zhi = yhi_ref[...] + ahi0_ref[...] + ahi1_ref[...] + b_ref[:, hh:]
        z = jnp.maximum(jnp.concatenate([zlo, zhi], axis=1), 0.0)
        w = jnp.dot(w3_ref[...], wl_ref[...], preferred_element_type=jnp.float32)
        o_ref[...] = jnp.dot(z, w, preferred_element_type=jnp.float32)

    hspec = pl.BlockSpec((_BN, hh), lambda i: (i, 0))
    return pl.pallas_call(
        body,
        grid=(n // _BN,),
        in_specs=[hspec, hspec, hspec, hspec, hspec, hspec,
                  pl.BlockSpec((1, 2 * hh), lambda i: (0, 0)),
                  pl.BlockSpec((2 * hh, l), lambda i: (0, 0)),
                  pl.BlockSpec((l, l), lambda i: (0, 0))],
        out_specs=pl.BlockSpec((_BN, l), lambda i: (i, 0)),
        out_shape=jax.ShapeDtypeStruct((n, l), jnp.float32),
    )(ylo, yhi, alo0, alo1, ahi0, ahi1, b, w3, wl)


def _tc_final(y, a0, a1, b3, wl, bl):
    n, l = y.shape

    def body(y_ref, a0_ref, a1_ref, b3_ref, wl_ref, bl_ref, o_ref):
        bp = jnp.dot(b3_ref[...], wl_ref[...],
                     preferred_element_type=jnp.float32) + bl_ref[...]
        o_ref[...] = y_ref[...] + a0_ref[...] + a1_ref[...] + bp

    return pl.pallas_call(
        body,
        grid=(n // _BN,),
        in_specs=[pl.BlockSpec((_BN, l), lambda i: (i, 0)),
                  pl.BlockSpec((_BN, l), lambda i: (i, 0)),
                  pl.BlockSpec((_BN, l), lambda i: (i, 0)),
                  pl.BlockSpec((1, l), lambda i: (0, 0)),
                  pl.BlockSpec((l, l), lambda i: (0, 0)),
                  pl.BlockSpec((1, l), lambda i: (0, 0))],
        out_specs=pl.BlockSpec((_BN, l), lambda i: (i, 0)),
        out_shape=jax.ShapeDtypeStruct((n, l), jnp.float32),
    )(y, a0, a1, b3, wl, bl)


def kernel(x, edge_index, batch, W1, b1, W2, b2, W3, b3, Wl, bl):
    n, d = x.shape
    e = edge_index.shape[1]
    src = edge_index[0]
    dst = edge_index[1]

    per_tile = -(-e // NW)
    per_tile = -(-per_tile // (2 * CH)) * (2 * CH)
    n_chunks = per_tile // CH
    e_pad = NW * per_tile
    n_pad = -(-n // (NS * 8)) * (NS * 8)
    src_p = jnp.concatenate([src, jnp.zeros((e_pad - e,), jnp.int32)])
    dst_p = jnp.concatenate([dst, jnp.full((e_pad - e,), n, jnp.int32)])
    src3 = src_p.reshape(NW, n_chunks, CH)
    dst3 = dst_p.reshape(NW, n_chunks, CH)

    x_p = jnp.concatenate([x, jnp.zeros((n_pad - n, d), jnp.float32)])

    rows_per_tile = n_pad // NS
    zeros_hh = jnp.zeros((rows_per_tile, W1.shape[1] // 2), jnp.float32)
    zeros_l = jnp.zeros((rows_per_tile, Wl.shape[0]), jnp.float32)

    b1r = b1.reshape(1, -1)
    b2r = b2.reshape(1, -1)
    b3r = b3.reshape(1, -1)
    blr = bl.reshape(1, -1)

    y1lo, y1hi = _tc_matmul2(x_p, W1)
    a1lo, a1hi = _sc_layer(y1lo, y1hi, src3, dst3, zeros_hh,
                           n_chunks=n_chunks, n_pad=n_pad)
    y2lo, y2hi = _tc_fuse3(y1lo, y1hi, a1lo[0], a1lo[1], a1hi[0], a1hi[1],
                           b1r, W2)
    a2lo, a2hi = _sc_layer(y2lo, y2hi, src3, dst3, zeros_hh,
                           n_chunks=n_chunks, n_pad=n_pad)
    y3 = _tc_fuse4(y2lo, y2hi, a2lo[0], a2lo[1], a2hi[0], a2hi[1],
                   b2r, W3, Wl)
    a3 = _sc_half(y3, src3, dst3, zeros_l, n_chunks=n_chunks, n_pad=n_pad)
    return _tc_final(y3, a3[0], a3[1], b3r, Wl, blr)[:n]
